# baseline (speedup 1.0000x reference)
import jax
import jax.numpy as jnp
from jax.experimental import pallas as pl
from jax.experimental.pallas import tpu as pltpu

_EDGE_BLK = 8192
_UNROLL = 512
_NBUF = 8
_NODE_TILE = 512

_ARBITRARY = pltpu.GridDimensionSemantics.ARBITRARY


def _spmm_kernel(ed_ref, w_ref, out_ref, a0, a1, a2, a3, a4, a5, a6, a7):
    accs = (a0, a1, a2, a3, a4, a5, a6, a7)

    @pl.when(pl.program_id(0) == 0)
    def _init():
        for a in accs:
            a[...] = jnp.zeros_like(a)

    def body(i, carry):
        base = i * _UNROLL
        for u in range(_UNROLL):
            e = base + u
            w = ed_ref[0, 0, e]
            s = w & 0xFFFF
            d = jax.lax.shift_right_logical(w, 16)
            a = accs[u % _NBUF]
            a[d] = a[d] + w_ref[s]
        return carry

    jax.lax.fori_loop(0, _EDGE_BLK // _UNROLL, body, 0)

    @pl.when(pl.program_id(0) == pl.num_programs(0) - 1)
    def _finish():
        n = out_ref.shape[0]
        s01 = (a0[...] + a1[...]) + (a2[...] + a3[...])
        s23 = (a4[...] + a5[...]) + (a6[...] + a7[...])
        out_ref[...] = (s01 + s23)[:n]


def _dense_kernel(oe_ref, x_ref, wl_ref, emb_ref,
                  wnx_ref, wne_ref, bn_ref,
                  wc1_ref, bc1_ref, wc2_ref, bc2_ref,
                  wf_ref, bf_ref, be_ref, y_ref):
    f32 = jnp.float32
    bf16 = jnp.bfloat16
    oe = oe_ref[...] + be_ref[...]
    out = oe + jnp.dot(oe.astype(bf16), wc1_ref[...],
                       preferred_element_type=f32) + bc1_ref[...]

    wl = wl_ref[...]
    t = wl.shape[0]
    nw = emb_ref.shape[0]
    ids = jax.lax.broadcasted_iota(jnp.int32, (t, nw), 1)
    emb = jnp.dot((ids == wl).astype(bf16), emb_ref[...],
                  preferred_element_type=f32)

    xh = (jnp.dot(x_ref[...], wnx_ref[...], preferred_element_type=f32)
          + jnp.dot(emb.astype(bf16), wne_ref[...], preferred_element_type=f32)
          + bn_ref[...])
    out = out + xh
    out = out + jnp.dot(xh.astype(bf16), wc2_ref[...],
                        preferred_element_type=f32) + bc2_ref[...]
    out = jnp.maximum(out, 0.0)
    y_ref[...] = (jnp.dot(out.astype(bf16), wf_ref[...],
                          preferred_element_type=f32) + bf_ref[...])


def kernel(w_edge, b_edge, wl_emb, w_node_x, w_node_e, b_node,
           w_cat1, b_cat1, w_cat2, b_cat2, w_final, b_final,
           edge_index, wl_indices, x):
    n, h = w_edge.shape
    f = x.shape[1]
    o = w_final.shape[1]
    nw, d = wl_emb.shape
    bf16 = jnp.bfloat16

    src = edge_index[0].astype(jnp.int32)
    dst = edge_index[1].astype(jnp.int32)
    e = src.shape[0]
    tiles = -(-e // _EDGE_BLK)
    pad = tiles * _EDGE_BLK - e
    if pad:
        src = jnp.concatenate([src, jnp.zeros((pad,), jnp.int32)])
        dst = jnp.concatenate([dst, jnp.full((pad,), n, jnp.int32)])
    packed = (dst << 16) | src

    oe3 = pl.pallas_call(
        _spmm_kernel,
        out_shape=jax.ShapeDtypeStruct((n, 1, h), jnp.float32),
        grid=(tiles,),
        in_specs=[
            pl.BlockSpec((1, 1, _EDGE_BLK), lambda i: (i, 0, 0),
                         memory_space=pltpu.SMEM),
            pl.BlockSpec((n, 1, h), lambda i: (0, 0, 0)),
        ],
        out_specs=pl.BlockSpec((n, 1, h), lambda i: (0, 0, 0)),
        scratch_shapes=[pltpu.VMEM((n + 8, 1, h), jnp.float32)
                        for _ in range(_NBUF)],
        compiler_params=pltpu.CompilerParams(
            dimension_semantics=(_ARBITRARY,),
            disable_bounds_checks=True,
            vmem_limit_bytes=60 * 1024 * 1024),
    )(packed.reshape(tiles, 1, _EDGE_BLK), w_edge[:, None, :])
    oe = oe3.reshape(n, h)

    nt = _NODE_TILE
    ntiles = n // nt
    full = lambda i: (0, 0)
    row = lambda i: (i, 0)

    y = pl.pallas_call(
        _dense_kernel,
        out_shape=jax.ShapeDtypeStruct((n, o), jnp.float32),
        grid=(ntiles,),
        in_specs=[
            pl.BlockSpec((nt, h), row),
            pl.BlockSpec((nt, f), row),
            pl.BlockSpec((nt, 1), row),
            pl.BlockSpec((nw, d), full),
            pl.BlockSpec((f, h), full),
            pl.BlockSpec((d, h), full),
            pl.BlockSpec((1, h), full),
            pl.BlockSpec((h, h), full),
            pl.BlockSpec((1, h), full),
            pl.BlockSpec((h, h), full),
            pl.BlockSpec((1, h), full),
            pl.BlockSpec((h, o), full),
            pl.BlockSpec((1, o), full),
            pl.BlockSpec((1, h), full),
        ],
        out_specs=pl.BlockSpec((nt, o), row),
        compiler_params=pltpu.CompilerParams(
            dimension_semantics=(_ARBITRARY,)),
    )(oe, x.astype(bf16), wl_indices.astype(jnp.int32)[:, None],
      wl_emb.astype(bf16),
      w_node_x.astype(bf16), w_node_e.astype(bf16), b_node[None, :],
      w_cat1.astype(bf16), b_cat1[None, :],
      w_cat2.astype(bf16), b_cat2[None, :],
      w_final.astype(bf16), b_final[None, :],
      b_edge[None, :])
    return y


# speedup vs baseline: 1.0157x; 1.0157x over previous
import jax
import jax.numpy as jnp
from jax.experimental import pallas as pl
from jax.experimental.pallas import tpu as pltpu

_EDGE_BLK = 4096
_UNROLL = 512
_NBUF = 4
_NODE_TILE = 512

_ARBITRARY = pltpu.GridDimensionSemantics.ARBITRARY


def _spmm_kernel(ed_ref, w_ref, out_ref, a0, a1, a2, a3):
    accs = (a0, a1, a2, a3)

    @pl.when(pl.program_id(0) == 0)
    def _init():
        for a in accs:
            a[...] = jnp.zeros_like(a)

    def body(i, carry):
        base = i * _UNROLL
        for u in range(_UNROLL):
            e = base + u
            w = ed_ref[0, 0, e]
            s = w & 0xFFFF
            d = jax.lax.shift_right_logical(w, 16)
            a = accs[u % _NBUF]
            a[d] = a[d] + w_ref[s]
        return carry

    jax.lax.fori_loop(0, _EDGE_BLK // _UNROLL, body, 0)

    @pl.when(pl.program_id(0) == pl.num_programs(0) - 1)
    def _finish():
        n = out_ref.shape[0]
        out_ref[...] = ((a0[...] + a1[...]) + (a2[...] + a3[...]))[:n]


def _dense_kernel(oe_ref, x_ref, wl_ref, emb_ref,
                  wnx_ref, wne_ref, bn_ref,
                  wc1_ref, bc1_ref, wc2_ref, bc2_ref,
                  wf_ref, bf_ref, be_ref, y_ref):
    f32 = jnp.float32
    bf16 = jnp.bfloat16
    oe = oe_ref[...] + be_ref[...]
    out = oe + jnp.dot(oe.astype(bf16), wc1_ref[...],
                       preferred_element_type=f32) + bc1_ref[...]

    wl = wl_ref[...]
    t = wl.shape[0]
    nw = emb_ref.shape[0]
    ids = jax.lax.broadcasted_iota(jnp.int32, (t, nw), 1)
    emb = jnp.dot((ids == wl).astype(bf16), emb_ref[...],
                  preferred_element_type=f32)

    xh = (jnp.dot(x_ref[...], wnx_ref[...], preferred_element_type=f32)
          + jnp.dot(emb.astype(bf16), wne_ref[...], preferred_element_type=f32)
          + bn_ref[...])
    out = out + xh
    out = out + jnp.dot(xh.astype(bf16), wc2_ref[...],
                        preferred_element_type=f32) + bc2_ref[...]
    out = jnp.maximum(out, 0.0)
    y_ref[...] = (jnp.dot(out.astype(bf16), wf_ref[...],
                          preferred_element_type=f32) + bf_ref[...])


def kernel(w_edge, b_edge, wl_emb, w_node_x, w_node_e, b_node,
           w_cat1, b_cat1, w_cat2, b_cat2, w_final, b_final,
           edge_index, wl_indices, x):
    n, h = w_edge.shape
    f = x.shape[1]
    o = w_final.shape[1]
    nw, d = wl_emb.shape
    bf16 = jnp.bfloat16

    src = edge_index[0].astype(jnp.int32)
    dst = edge_index[1].astype(jnp.int32)
    e = src.shape[0]
    tiles = -(-e // _EDGE_BLK)
    pad = tiles * _EDGE_BLK - e
    if pad:
        src = jnp.concatenate([src, jnp.zeros((pad,), jnp.int32)])
        dst = jnp.concatenate([dst, jnp.full((pad,), n, jnp.int32)])
    packed = (dst << 16) | src

    oe3 = pl.pallas_call(
        _spmm_kernel,
        out_shape=jax.ShapeDtypeStruct((n, 1, h), jnp.float32),
        grid=(tiles,),
        in_specs=[
            pl.BlockSpec((1, 1, _EDGE_BLK), lambda i: (i, 0, 0),
                         memory_space=pltpu.SMEM),
            pl.BlockSpec((n, 1, h), lambda i: (0, 0, 0)),
        ],
        out_specs=pl.BlockSpec((n, 1, h), lambda i: (0, 0, 0)),
        scratch_shapes=[pltpu.VMEM((n + 8, 1, h), jnp.float32)
                        for _ in range(_NBUF)],
        compiler_params=pltpu.CompilerParams(
            dimension_semantics=(_ARBITRARY,),
            disable_bounds_checks=True,
            vmem_limit_bytes=60 * 1024 * 1024),
    )(packed.reshape(tiles, 1, _EDGE_BLK), w_edge[:, None, :])
    oe = oe3.reshape(n, h)

    nt = _NODE_TILE
    ntiles = n // nt
    full = lambda i: (0, 0)
    row = lambda i: (i, 0)

    y = pl.pallas_call(
        _dense_kernel,
        out_shape=jax.ShapeDtypeStruct((n, o), jnp.float32),
        grid=(ntiles,),
        in_specs=[
            pl.BlockSpec((nt, h), row),
            pl.BlockSpec((nt, f), row),
            pl.BlockSpec((nt, 1), row),
            pl.BlockSpec((nw, d), full),
            pl.BlockSpec((f, h), full),
            pl.BlockSpec((d, h), full),
            pl.BlockSpec((1, h), full),
            pl.BlockSpec((h, h), full),
            pl.BlockSpec((1, h), full),
            pl.BlockSpec((h, h), full),
            pl.BlockSpec((1, h), full),
            pl.BlockSpec((h, o), full),
            pl.BlockSpec((1, o), full),
            pl.BlockSpec((1, h), full),
        ],
        out_specs=pl.BlockSpec((nt, o), row),
        compiler_params=pltpu.CompilerParams(
            dimension_semantics=(_ARBITRARY,)),
    )(oe, x.astype(bf16), wl_indices.astype(jnp.int32)[:, None],
      wl_emb.astype(bf16),
      w_node_x.astype(bf16), w_node_e.astype(bf16), b_node[None, :],
      w_cat1.astype(bf16), b_cat1[None, :],
      w_cat2.astype(bf16), b_cat2[None, :],
      w_final.astype(bf16), b_final[None, :],
      b_edge[None, :])
    return y


# speedup vs baseline: 1.0158x; 1.0001x over previous
import jax
import jax.numpy as jnp
from jax.experimental import pallas as pl
from jax.experimental.pallas import tpu as pltpu

_EDGE_BLK = 2048
_UNROLL = 512
_NBUF = 4
_NODE_TILE = 512

_ARBITRARY = pltpu.GridDimensionSemantics.ARBITRARY


def _spmm_kernel(ed_ref, w_ref, out_ref, a0, a1, a2, a3):
    accs = (a0, a1, a2, a3)

    @pl.when(pl.program_id(0) == 0)
    def _init():
        for a in accs:
            a[...] = jnp.zeros_like(a)

    def body(i, carry):
        base = i * _UNROLL
        for u in range(_UNROLL):
            e = base + u
            w = ed_ref[0, 0, e]
            s = w & 0xFFFF
            d = jax.lax.shift_right_logical(w, 16)
            a = accs[u % _NBUF]
            a[d] = a[d] + w_ref[s]
        return carry

    jax.lax.fori_loop(0, _EDGE_BLK // _UNROLL, body, 0)

    @pl.when(pl.program_id(0) == pl.num_programs(0) - 1)
    def _finish():
        n = out_ref.shape[0]
        out_ref[...] = ((a0[...] + a1[...]) + (a2[...] + a3[...]))[:n]


def _dense_kernel(oe_ref, x_ref, wl_ref, emb_ref,
                  wnx_ref, wne_ref, bn_ref,
                  wc1_ref, bc1_ref, wc2_ref, bc2_ref,
                  wf_ref, bf_ref, be_ref, y_ref):
    f32 = jnp.float32
    bf16 = jnp.bfloat16
    oe = oe_ref[...] + be_ref[...]
    out = oe + jnp.dot(oe.astype(bf16), wc1_ref[...],
                       preferred_element_type=f32) + bc1_ref[...]

    wl = wl_ref[...]
    t = wl.shape[0]
    nw = emb_ref.shape[0]
    ids = jax.lax.broadcasted_iota(jnp.int32, (t, nw), 1)
    emb = jnp.dot((ids == wl).astype(bf16), emb_ref[...],
                  preferred_element_type=f32)

    xh = (jnp.dot(x_ref[...], wnx_ref[...], preferred_element_type=f32)
          + jnp.dot(emb.astype(bf16), wne_ref[...], preferred_element_type=f32)
          + bn_ref[...])
    out = out + xh
    out = out + jnp.dot(xh.astype(bf16), wc2_ref[...],
                        preferred_element_type=f32) + bc2_ref[...]
    out = jnp.maximum(out, 0.0)
    y_ref[...] = (jnp.dot(out.astype(bf16), wf_ref[...],
                          preferred_element_type=f32) + bf_ref[...])


def kernel(w_edge, b_edge, wl_emb, w_node_x, w_node_e, b_node,
           w_cat1, b_cat1, w_cat2, b_cat2, w_final, b_final,
           edge_index, wl_indices, x):
    n, h = w_edge.shape
    f = x.shape[1]
    o = w_final.shape[1]
    nw, d = wl_emb.shape
    bf16 = jnp.bfloat16

    src = edge_index[0].astype(jnp.int32)
    dst = edge_index[1].astype(jnp.int32)
    e = src.shape[0]
    tiles = -(-e // _EDGE_BLK)
    pad = tiles * _EDGE_BLK - e
    if pad:
        src = jnp.concatenate([src, jnp.zeros((pad,), jnp.int32)])
        dst = jnp.concatenate([dst, jnp.full((pad,), n, jnp.int32)])
    packed = (dst << 16) | src

    oe3 = pl.pallas_call(
        _spmm_kernel,
        out_shape=jax.ShapeDtypeStruct((n, 1, h), jnp.float32),
        grid=(tiles,),
        in_specs=[
            pl.BlockSpec((1, 1, _EDGE_BLK), lambda i: (i, 0, 0),
                         memory_space=pltpu.SMEM),
            pl.BlockSpec((n, 1, h), lambda i: (0, 0, 0)),
        ],
        out_specs=pl.BlockSpec((n, 1, h), lambda i: (0, 0, 0)),
        scratch_shapes=[pltpu.VMEM((n + 8, 1, h), jnp.float32)
                        for _ in range(_NBUF)],
        compiler_params=pltpu.CompilerParams(
            dimension_semantics=(_ARBITRARY,),
            disable_bounds_checks=True,
            vmem_limit_bytes=60 * 1024 * 1024),
    )(packed.reshape(tiles, 1, _EDGE_BLK), w_edge[:, None, :])
    oe = oe3.reshape(n, h)

    nt = _NODE_TILE
    ntiles = n // nt
    full = lambda i: (0, 0)
    row = lambda i: (i, 0)

    y = pl.pallas_call(
        _dense_kernel,
        out_shape=jax.ShapeDtypeStruct((n, o), jnp.float32),
        grid=(ntiles,),
        in_specs=[
            pl.BlockSpec((nt, h), row),
            pl.BlockSpec((nt, f), row),
            pl.BlockSpec((nt, 1), row),
            pl.BlockSpec((nw, d), full),
            pl.BlockSpec((f, h), full),
            pl.BlockSpec((d, h), full),
            pl.BlockSpec((1, h), full),
            pl.BlockSpec((h, h), full),
            pl.BlockSpec((1, h), full),
            pl.BlockSpec((h, h), full),
            pl.BlockSpec((1, h), full),
            pl.BlockSpec((h, o), full),
            pl.BlockSpec((1, o), full),
            pl.BlockSpec((1, h), full),
        ],
        out_specs=pl.BlockSpec((nt, o), row),
        compiler_params=pltpu.CompilerParams(
            dimension_semantics=(_ARBITRARY,)),
    )(oe, x.astype(bf16), wl_indices.astype(jnp.int32)[:, None],
      wl_emb.astype(bf16),
      w_node_x.astype(bf16), w_node_e.astype(bf16), b_node[None, :],
      w_cat1.astype(bf16), b_cat1[None, :],
      w_cat2.astype(bf16), b_cat2[None, :],
      w_final.astype(bf16), b_final[None, :],
      b_edge[None, :])
    return y
